# baseline (speedup 1.0000x reference)
import functools

import jax
import jax.numpy as jnp
from jax import lax
from jax.experimental import pallas as pl
from jax.experimental.pallas import tpu as pltpu
from jax.experimental.pallas import tpu_sc as plsc

LON_MIN, LON_MAX = 36.838, 70.092
LAT_MIN, LAT_MAX = -10.474, 31.586
BINS = 100
EMBED_DIM = 128
L = 16
GCHUNK = 128
NBUF = 4

LAT_INV = float((BINS - 2) / (LAT_MAX - LAT_MIN))
LON_INV = float((BINS - 2) / (LON_MAX - LON_MIN))


def _digitize16(x, bins_ref, b0, inv, off):
    t = (x - b0) * inv
    t = jnp.minimum(jnp.maximum(t, -1.0), float(BINS))
    g = jnp.minimum(jnp.maximum(t.astype(jnp.int32) + 1, 0), BINS - 1)
    lo = plsc.load_gather(bins_ref, [g + off])
    hi = plsc.load_gather(bins_ref, [g + off + 1])
    one = jnp.full((L,), 1, jnp.int32)
    zero = jnp.full((L,), 0, jnp.int32)
    idx = g - 1 + jnp.where(x >= lo, one, zero) + jnp.where(x >= hi, one, zero)
    return jnp.minimum(jnp.maximum(idx, 0), BINS - 1)


@functools.partial(jax.jit, static_argnames=("batch",))
def _sc_embed(lon, lat, lat_table, lon_table, bins_pad, *, batch):
    info = plsc.get_sparse_core_info()
    nw = info.num_cores * info.num_subcores
    rows_w = 2 * batch // nw
    elems_w = batch // nw
    nchunks = rows_w // GCHUNK
    mesh = plsc.VectorSubcoreMesh(core_axis_name="c", subcore_axis_name="s")

    @functools.partial(
        pl.kernel,
        out_type=jax.ShapeDtypeStruct((2 * batch, EMBED_DIM), jnp.float32),
        mesh=mesh,
        scratch_types=[
            pltpu.VMEM((elems_w,), jnp.float32),
            pltpu.VMEM((elems_w,), jnp.float32),
            pltpu.VMEM_SHARED((2 * BINS, EMBED_DIM), jnp.float32),
            pltpu.VMEM((256,), jnp.float32),
            pltpu.VMEM((nchunks, GCHUNK), jnp.int32),
            pltpu.VMEM((NBUF, GCHUNK, EMBED_DIM), jnp.float32),
            pltpu.SemaphoreType.DMA,
            pltpu.SemaphoreType.DMA,
            pltpu.SemaphoreType.DMA,
        ],
        compiler_params=pltpu.CompilerParams(needs_layout_passes=False),
    )
    def k(lon_hbm, lat_hbm, lat_t_hbm, lon_t_hbm, bins_hbm, out_hbm,
          lon_v, lat_v, tbl_v, bins_v, idx_v, bufs, insem, gsem, wsem):
        wid = lax.axis_index("s") * info.num_cores + lax.axis_index("c")
        cin = []
        for src, dst in ((lon_hbm, lon_v), (lat_hbm, lat_v)):
            c = pltpu.make_async_copy(
                src.at[pl.ds(wid * elems_w, elems_w)], dst, insem)
            c.start()
            cin.append(c)
        cb = pltpu.make_async_copy(bins_hbm, bins_v, insem)
        cb.start()
        cin.append(cb)

        @pl.when(lax.axis_index("s") == 0)
        def _stage_lat():
            pltpu.sync_copy(lat_t_hbm, tbl_v.at[pl.ds(0, BINS)])

        @pl.when(lax.axis_index("s") == 1)
        def _stage_lon():
            pltpu.sync_copy(lon_t_hbm, tbl_v.at[pl.ds(BINS, BINS)])

        for c in cin:
            c.wait()

        lane = lax.iota(jnp.int32, L)
        epc = GCHUNK // 2 // L

        def compute_idx(j):
            def body(cc, _):
                c = epc * j + cc
                i = L * c + lane
                lon_x = lon_v[pl.ds(L * c, L)]
                lat_x = lat_v[pl.ds(L * c, L)]
                lat_i = _digitize16(lat_x, bins_v, LAT_MIN, LAT_INV, 0)
                lon_i = _digitize16(lon_x, bins_v, LON_MIN, LON_INV, 128)
                q = ((i >> 3) << 4) + (i & 7)
                gsh = GCHUNK.bit_length() - 1
                plsc.store_scatter(idx_v, [q >> gsh, q & (GCHUNK - 1)], lat_i)
                plsc.store_scatter(idx_v,
                                   [(q + 8) >> gsh, (q + 8) & (GCHUNK - 1)],
                                   lon_i + BINS)
                return _

            lax.fori_loop(0, epc, body, None)

        def write_out(j):
            w = pltpu.make_async_copy(
                bufs.at[j % NBUF],
                out_hbm.at[pl.ds(wid * rows_w + GCHUNK * j, GCHUNK)], wsem)
            w.start()
            return w

        gathers = [None] * nchunks
        writes = [None] * nchunks
        for j in range(nchunks):
            compute_idx(j)
            if j == 0:
                plsc.subcore_barrier()
            if j >= NBUF:
                writes[j - NBUF].wait()
            g = pltpu.make_async_copy(tbl_v.at[idx_v.at[j]], bufs.at[j % NBUF],
                                      gsem)
            g.start()
            gathers[j] = g
            if j >= 1:
                gathers[j - 1].wait()
                writes[j - 1] = write_out(j - 1)
        gathers[nchunks - 1].wait()
        writes[nchunks - 1] = write_out(nchunks - 1)
        for j in range(max(0, nchunks - NBUF), nchunks):
            writes[j].wait()

    return k(lon, lat, lat_table, lon_table, bins_pad)


def kernel(crs, lat_table, lon_table):
    batch = crs.shape[0]
    lat_bins = jnp.linspace(LAT_MIN, LAT_MAX, BINS - 1)
    lon_bins = jnp.linspace(LON_MIN, LON_MAX, BINS - 1)
    ninf = jnp.float32(-jnp.inf)
    bins_pad = jnp.full((256,), jnp.inf, jnp.float32)
    bins_pad = bins_pad.at[0].set(ninf).at[1:BINS].set(lat_bins)
    bins_pad = bins_pad.at[128].set(ninf).at[129:128 + BINS].set(lon_bins)
    out = _sc_embed(crs[:, 0], crs[:, 1], lat_table, lon_table, bins_pad,
                    batch=batch)
    out = out.reshape(batch // 8, 2, 8, EMBED_DIM).transpose(0, 2, 1, 3)
    return out.reshape(batch, 2 * EMBED_DIM)
